# initial kernel scaffold (unmeasured)
import jax
import jax.numpy as jnp
from jax import lax
from jax.experimental import pallas as pl
from jax.experimental.pallas import tpu as pltpu

N_DEV = 8
B = 64
D = 2048
ROWS = B // N_DEV
J = 4


def _layer(x, Win, Wout, *, last, cid):
    _, hs = Win.shape
    bh = hs // J

    def body(x_ref, win_ref, wout_ref, out_ref,
             acc, sendbuf, rs_buf, rs_send, rs_recv,
             ag_src, ag_buf, ag_send, ag_recv):
        j = pl.program_id(0)
        my = lax.axis_index("i")

        @pl.when(j == 0)
        def _():
            acc[...] = jnp.zeros_like(acc)

        xv = x_ref[...].astype(jnp.bfloat16)
        h = jnp.dot(xv, win_ref[...].astype(jnp.bfloat16),
                    preferred_element_type=jnp.float32)
        h = jnp.maximum(h, 0.0).astype(jnp.bfloat16)
        acc[...] += jnp.dot(h, wout_ref[...].astype(jnp.bfloat16),
                            preferred_element_type=jnp.float32)

        @pl.when(j == J - 1)
        def _():
            sendbuf[...] = acc[...].astype(jnp.bfloat16).reshape(N_DEV, ROWS, D)
            rs_sends = []
            for d in range(1, N_DEV):
                peer = (my + d) % N_DEV
                rdma = pltpu.make_async_remote_copy(
                    src_ref=sendbuf.at[peer],
                    dst_ref=rs_buf.at[my],
                    send_sem=rs_send.at[peer],
                    recv_sem=rs_recv.at[my],
                    device_id=(peer,),
                    device_id_type=pl.DeviceIdType.MESH,
                )
                rdma.start()
                rs_sends.append(rdma)
            for d in range(1, N_DEV):
                s = (my + d) % N_DEV
                recv = pltpu.make_async_remote_copy(
                    src_ref=sendbuf.at[s],
                    dst_ref=rs_buf.at[s],
                    send_sem=rs_send.at[s],
                    recv_sem=rs_recv.at[s],
                    device_id=(s,),
                    device_id_type=pl.DeviceIdType.MESH,
                )
                recv.wait_recv()
            for rdma in rs_sends:
                rdma.wait_send()

            sid = lax.broadcasted_iota(jnp.int32, (N_DEV, 1, 1), 0)
            slots = jnp.where(sid == my, 0.0, rs_buf[...].astype(jnp.float32))
            red = jnp.sum(slots, axis=0) + acc[...].reshape(N_DEV, ROWS, D)[my]

            if last:
                out_ref[...] = red
            else:
                ag_src[...] = red.astype(jnp.bfloat16)
                ag_sends = []
                for d in range(1, N_DEV):
                    peer = (my + d) % N_DEV
                    rdma = pltpu.make_async_remote_copy(
                        src_ref=ag_src,
                        dst_ref=ag_buf.at[my],
                        send_sem=ag_send.at[peer],
                        recv_sem=ag_recv.at[my],
                        device_id=(peer,),
                        device_id_type=pl.DeviceIdType.MESH,
                    )
                    rdma.start()
                    ag_sends.append(rdma)
                for d in range(1, N_DEV):
                    s = (my + d) % N_DEV
                    recv = pltpu.make_async_remote_copy(
                        src_ref=ag_src,
                        dst_ref=ag_buf.at[s],
                        send_sem=ag_send.at[s],
                        recv_sem=ag_recv.at[s],
                        device_id=(s,),
                        device_id_type=pl.DeviceIdType.MESH,
                    )
                    recv.wait_recv()
                for rdma in ag_sends:
                    rdma.wait_send()
                gathered = jnp.where(sid == my, ag_src[...][None, :, :],
                                     ag_buf[...])
                out_ref[...] = gathered.reshape(B, D)

    if last:
        out_shape = jax.ShapeDtypeStruct((ROWS, D), jnp.float32)
        out_spec = pl.BlockSpec((ROWS, D), lambda j: (0, 0))
    else:
        out_shape = jax.ShapeDtypeStruct((B, D), jnp.bfloat16)
        out_spec = pl.BlockSpec((B, D), lambda j: (0, 0))

    return pl.pallas_call(
        body,
        grid=(J,),
        in_specs=[
            pl.BlockSpec((B, D), lambda j: (0, 0)),
            pl.BlockSpec((D, bh), lambda j: (0, j)),
            pl.BlockSpec((bh, D), lambda j: (j, 0)),
        ],
        out_specs=out_spec,
        out_shape=out_shape,
        scratch_shapes=[
            pltpu.VMEM((B, D), jnp.float32),
            pltpu.VMEM((N_DEV, ROWS, D), jnp.bfloat16),
            pltpu.VMEM((N_DEV, ROWS, D), jnp.bfloat16),
            pltpu.SemaphoreType.DMA((N_DEV,)),
            pltpu.SemaphoreType.DMA((N_DEV,)),
            pltpu.VMEM((ROWS, D), jnp.bfloat16),
            pltpu.VMEM((N_DEV, ROWS, D), jnp.bfloat16),
            pltpu.SemaphoreType.DMA((N_DEV,)),
            pltpu.SemaphoreType.DMA((N_DEV,)),
        ],
        compiler_params=pltpu.CompilerParams(collective_id=cid),
    )(x, Win, Wout)


def kernel(x, Win0, Wout0, Win1, Wout1, Win2, Wout2):
    x1 = _layer(x, Win0, Wout0, last=False, cid=0)
    x2 = _layer(x1, Win1, Wout1, last=False, cid=1)
    return _layer(x2, Win2, Wout2, last=True, cid=2)


# baseline (device time: 104512 ns/iter reference)
import jax
import jax.numpy as jnp
from jax import lax
from jax.experimental import pallas as pl
from jax.experimental.pallas import tpu as pltpu

N_DEV = 8
B = 64
D = 2048
ROWS = B // N_DEV
J = 8


def _layer(x, Win, Wout, *, last, cid):
    _, hs = Win.shape
    bh = hs // J

    def body(x_ref, win_ref, wout_ref, out_ref,
             acc, sendbuf, rs_buf, rs_send, rs_recv,
             ag_src, ag_buf, ag_send, ag_recv):
        j = pl.program_id(0)
        my = lax.axis_index("i")

        @pl.when(j == 0)
        def _():
            acc[...] = jnp.zeros_like(acc)

        xv = x_ref[...].astype(jnp.bfloat16)
        h = jnp.dot(xv, win_ref[...].astype(jnp.bfloat16),
                    preferred_element_type=jnp.float32)
        h = jnp.maximum(h, 0.0).astype(jnp.bfloat16)
        acc[...] += jnp.dot(h, wout_ref[...].astype(jnp.bfloat16),
                            preferred_element_type=jnp.float32)

        @pl.when(j == J - 1)
        def _():
            sendbuf[...] = acc[...].astype(jnp.bfloat16).reshape(N_DEV, ROWS, D)
            rs_sends = []
            for d in range(1, N_DEV):
                peer = (my + d) % N_DEV
                rdma = pltpu.make_async_remote_copy(
                    src_ref=sendbuf.at[peer],
                    dst_ref=rs_buf.at[my],
                    send_sem=rs_send.at[peer],
                    recv_sem=rs_recv.at[my],
                    device_id=(peer,),
                    device_id_type=pl.DeviceIdType.MESH,
                )
                rdma.start()
                rs_sends.append(rdma)
            for d in range(1, N_DEV):
                s = (my + d) % N_DEV
                recv = pltpu.make_async_remote_copy(
                    src_ref=sendbuf.at[s],
                    dst_ref=rs_buf.at[s],
                    send_sem=rs_send.at[s],
                    recv_sem=rs_recv.at[s],
                    device_id=(s,),
                    device_id_type=pl.DeviceIdType.MESH,
                )
                recv.wait_recv()
            for rdma in rs_sends:
                rdma.wait_send()

            sid = lax.broadcasted_iota(jnp.int32, (N_DEV, 1, 1), 0)
            slots = jnp.where(sid == my, 0.0, rs_buf[...].astype(jnp.float32))
            red = jnp.sum(slots, axis=0) + acc[pl.ds(my * ROWS, ROWS), :]

            if last:
                out_ref[...] = red
            else:
                ag_src[...] = red.astype(jnp.bfloat16)
                ag_sends = []
                for d in range(1, N_DEV):
                    peer = (my + d) % N_DEV
                    rdma = pltpu.make_async_remote_copy(
                        src_ref=ag_src,
                        dst_ref=ag_buf.at[my],
                        send_sem=ag_send.at[peer],
                        recv_sem=ag_recv.at[my],
                        device_id=(peer,),
                        device_id_type=pl.DeviceIdType.MESH,
                    )
                    rdma.start()
                    ag_sends.append(rdma)
                for d in range(1, N_DEV):
                    s = (my + d) % N_DEV
                    recv = pltpu.make_async_remote_copy(
                        src_ref=ag_src,
                        dst_ref=ag_buf.at[s],
                        send_sem=ag_send.at[s],
                        recv_sem=ag_recv.at[s],
                        device_id=(s,),
                        device_id_type=pl.DeviceIdType.MESH,
                    )
                    recv.wait_recv()
                for rdma in ag_sends:
                    rdma.wait_send()
                gathered = jnp.where(sid == my, ag_src[...][None, :, :],
                                     ag_buf[...])
                out_ref[...] = gathered.reshape(B, D)

    if last:
        out_shape = jax.ShapeDtypeStruct((ROWS, D), jnp.float32)
        out_spec = pl.BlockSpec((ROWS, D), lambda j: (0, 0))
    else:
        out_shape = jax.ShapeDtypeStruct((B, D), jnp.bfloat16)
        out_spec = pl.BlockSpec((B, D), lambda j: (0, 0))

    return pl.pallas_call(
        body,
        grid=(J,),
        in_specs=[
            pl.BlockSpec((B, D), lambda j: (0, 0)),
            pl.BlockSpec((D, bh), lambda j: (0, j)),
            pl.BlockSpec((bh, D), lambda j: (j, 0)),
        ],
        out_specs=out_spec,
        out_shape=out_shape,
        scratch_shapes=[
            pltpu.VMEM((B, D), jnp.float32),
            pltpu.VMEM((N_DEV, ROWS, D), jnp.bfloat16),
            pltpu.VMEM((N_DEV, ROWS, D), jnp.bfloat16),
            pltpu.SemaphoreType.DMA((N_DEV,)),
            pltpu.SemaphoreType.DMA((N_DEV,)),
            pltpu.VMEM((ROWS, D), jnp.bfloat16),
            pltpu.VMEM((N_DEV, ROWS, D), jnp.bfloat16),
            pltpu.SemaphoreType.DMA((N_DEV,)),
            pltpu.SemaphoreType.DMA((N_DEV,)),
        ],
    )(x, Win, Wout)


def kernel(x, Win0, Wout0, Win1, Wout1, Win2, Wout2):
    x1 = _layer(x, Win0, Wout0, last=False, cid=0)
    x2 = _layer(x1, Win1, Wout1, last=False, cid=1)
    return _layer(x2, Win2, Wout2, last=True, cid=2)


# device time: 89627 ns/iter; 1.1661x vs baseline; 1.1661x over previous
import jax
import jax.numpy as jnp
from jax import lax
from jax.experimental import pallas as pl
from jax.experimental.pallas import tpu as pltpu

N_DEV = 8
B = 64
D = 2048
HS = 4096
ROWS = B // N_DEV
J = 8
BH = HS // J
NBUF = 3


def kernel(x, Win0, Wout0, Win1, Wout1, Win2, Wout2):
    def body(x_ref, win0, wout0, win1, wout1, win2, wout2, out_ref,
             wa, wb, wa_sem, wb_sem, xbuf, acc, sendbuf, rs_buf,
             rs_send, rs_recv, ag_send, ag_recv):
        my = lax.axis_index("i")
        wins = [win0, win1, win2]
        wouts = [wout0, wout1, wout2]
        T = 3 * J

        def fetch(t):
            k, j = divmod(t, J)
            s = t % NBUF
            ca = pltpu.make_async_copy(
                wins[k].at[:, pl.ds(j * BH, BH)], wa.at[s], wa_sem.at[s])
            cb = pltpu.make_async_copy(
                wouts[k].at[pl.ds(j * BH, BH), :], wb.at[s], wb_sem.at[s])
            return ca, cb

        def comm(k):
            sendbuf[...] = acc[...].astype(jnp.bfloat16)
            rs_sends = []
            for d in range(1, N_DEV):
                peer = (my + d) % N_DEV
                r = pltpu.make_async_remote_copy(
                    src_ref=sendbuf.at[pl.ds(peer * ROWS, ROWS), :],
                    dst_ref=rs_buf.at[my],
                    send_sem=rs_send.at[peer],
                    recv_sem=rs_recv.at[my],
                    device_id=(peer,),
                    device_id_type=pl.DeviceIdType.MESH,
                )
                r.start()
                rs_sends.append(r)
            for d in range(1, N_DEV):
                src = (my + d) % N_DEV
                pltpu.make_async_remote_copy(
                    src_ref=sendbuf.at[pl.ds(src * ROWS, ROWS), :],
                    dst_ref=rs_buf.at[src],
                    send_sem=rs_send.at[src],
                    recv_sem=rs_recv.at[src],
                    device_id=(src,),
                    device_id_type=pl.DeviceIdType.MESH,
                ).wait_recv()
            for r in rs_sends:
                r.wait_send()

            sid = lax.broadcasted_iota(jnp.int32, (N_DEV, 1, 1), 0)
            slots = jnp.where(sid == my, 0.0, rs_buf[...].astype(jnp.float32))
            red = jnp.sum(slots, axis=0) + acc[pl.ds(my * ROWS, ROWS), :]

            if k == 2:
                out_ref[...] = red
                return
            xbuf[pl.ds(my * ROWS, ROWS), :] = red.astype(jnp.bfloat16)
            ag_sends = []
            for d in range(1, N_DEV):
                peer = (my + d) % N_DEV
                r = pltpu.make_async_remote_copy(
                    src_ref=xbuf.at[pl.ds(my * ROWS, ROWS), :],
                    dst_ref=xbuf.at[pl.ds(my * ROWS, ROWS), :],
                    send_sem=ag_send.at[peer],
                    recv_sem=ag_recv.at[my],
                    device_id=(peer,),
                    device_id_type=pl.DeviceIdType.MESH,
                )
                r.start()
                ag_sends.append(r)
            for d in range(1, N_DEV):
                src = (my + d) % N_DEV
                pltpu.make_async_remote_copy(
                    src_ref=xbuf.at[pl.ds(src * ROWS, ROWS), :],
                    dst_ref=xbuf.at[pl.ds(src * ROWS, ROWS), :],
                    send_sem=ag_send.at[src],
                    recv_sem=ag_recv.at[src],
                    device_id=(src,),
                    device_id_type=pl.DeviceIdType.MESH,
                ).wait_recv()
            for r in ag_sends:
                r.wait_send()

        for c in fetch(0) + fetch(1):
            c.start()
        xbuf[...] = x_ref[...].astype(jnp.bfloat16)

        for t in range(T):
            k, j = divmod(t, J)
            if t + 2 < T:
                for c in fetch(t + 2):
                    c.start()
            for c in fetch(t):
                c.wait()
            s = t % NBUF
            h = jnp.dot(xbuf[...], wa[s].astype(jnp.bfloat16),
                        preferred_element_type=jnp.float32)
            h = jnp.maximum(h, 0.0).astype(jnp.bfloat16)
            p = jnp.dot(h, wb[s].astype(jnp.bfloat16),
                        preferred_element_type=jnp.float32)
            if j == 0:
                acc[...] = p
            else:
                acc[...] += p
            if j == J - 1:
                comm(k)

    return pl.pallas_call(
        body,
        in_specs=[
            pl.BlockSpec(memory_space=pltpu.VMEM),
            pl.BlockSpec(memory_space=pl.ANY),
            pl.BlockSpec(memory_space=pl.ANY),
            pl.BlockSpec(memory_space=pl.ANY),
            pl.BlockSpec(memory_space=pl.ANY),
            pl.BlockSpec(memory_space=pl.ANY),
            pl.BlockSpec(memory_space=pl.ANY),
        ],
        out_specs=pl.BlockSpec(memory_space=pltpu.VMEM),
        out_shape=jax.ShapeDtypeStruct((ROWS, D), jnp.float32),
        scratch_shapes=[
            pltpu.VMEM((NBUF, D, BH), jnp.float32),
            pltpu.VMEM((NBUF, BH, D), jnp.float32),
            pltpu.SemaphoreType.DMA((NBUF,)),
            pltpu.SemaphoreType.DMA((NBUF,)),
            pltpu.VMEM((B, D), jnp.bfloat16),
            pltpu.VMEM((B, D), jnp.float32),
            pltpu.VMEM((B, D), jnp.bfloat16),
            pltpu.VMEM((N_DEV, ROWS, D), jnp.bfloat16),
            pltpu.SemaphoreType.DMA((N_DEV,)),
            pltpu.SemaphoreType.DMA((N_DEV,)),
            pltpu.SemaphoreType.DMA((N_DEV,)),
            pltpu.SemaphoreType.DMA((N_DEV,)),
        ],
    )(x, Win0, Wout0, Win1, Wout1, Win2, Wout2)


# device time: 65980 ns/iter; 1.5840x vs baseline; 1.3584x over previous
import jax
import jax.numpy as jnp
from jax import lax
from jax.experimental import pallas as pl
from jax.experimental.pallas import tpu as pltpu

import os
_NOCOMM = os.environ.get("KERNEL_NOCOMM") == "1"

N_DEV = 8
B = 64
D = 2048
HS = 4096
ROWS = B // N_DEV
J = 8
BH = HS // J
NBUF = 3


def kernel(x, Win0, Wout0, Win1, Wout1, Win2, Wout2):
    def body(x_ref, win0, wout0, win1, wout1, win2, wout2, out_ref,
             wa, wb, wa_sem, wb_sem, xbuf, acc, sendbuf, rs_buf,
             rs_send, rs_recv, ag_send, ag_recv):
        my = lax.axis_index("i")
        wins = [win0, win1, win2]
        wouts = [wout0, wout1, wout2]
        T = 3 * J

        def fetch(t):
            k, j = divmod(t, J)
            s = t % NBUF
            ca = pltpu.make_async_copy(
                wins[k].at[:, pl.ds(j * BH, BH)], wa.at[s], wa_sem.at[s])
            cb = pltpu.make_async_copy(
                wouts[k].at[pl.ds(j * BH, BH), :], wb.at[s], wb_sem.at[s])
            return ca, cb

        def comm(k):
            if _NOCOMM:
                if k == 2:
                    out_ref[...] = acc[pl.ds(my * ROWS, ROWS), :]
                return
            sendbuf[...] = acc[...].astype(jnp.bfloat16)
            rs_sends = []
            for d in range(1, N_DEV):
                peer = (my + d) % N_DEV
                r = pltpu.make_async_remote_copy(
                    src_ref=sendbuf.at[pl.ds(peer * ROWS, ROWS), :],
                    dst_ref=rs_buf.at[my],
                    send_sem=rs_send.at[peer],
                    recv_sem=rs_recv.at[my],
                    device_id=(peer,),
                    device_id_type=pl.DeviceIdType.MESH,
                )
                r.start()
                rs_sends.append(r)
            for d in range(1, N_DEV):
                src = (my + d) % N_DEV
                pltpu.make_async_remote_copy(
                    src_ref=sendbuf.at[pl.ds(src * ROWS, ROWS), :],
                    dst_ref=rs_buf.at[src],
                    send_sem=rs_send.at[src],
                    recv_sem=rs_recv.at[src],
                    device_id=(src,),
                    device_id_type=pl.DeviceIdType.MESH,
                ).wait_recv()
            for r in rs_sends:
                r.wait_send()

            sid = lax.broadcasted_iota(jnp.int32, (N_DEV, 1, 1), 0)
            slots = jnp.where(sid == my, 0.0, rs_buf[...].astype(jnp.float32))
            red = jnp.sum(slots, axis=0) + acc[pl.ds(my * ROWS, ROWS), :]

            if k == 2:
                out_ref[...] = red
                return
            xbuf[pl.ds(my * ROWS, ROWS), :] = red.astype(jnp.bfloat16)
            ag_sends = []
            for d in range(1, N_DEV):
                peer = (my + d) % N_DEV
                r = pltpu.make_async_remote_copy(
                    src_ref=xbuf.at[pl.ds(my * ROWS, ROWS), :],
                    dst_ref=xbuf.at[pl.ds(my * ROWS, ROWS), :],
                    send_sem=ag_send.at[peer],
                    recv_sem=ag_recv.at[my],
                    device_id=(peer,),
                    device_id_type=pl.DeviceIdType.MESH,
                )
                r.start()
                ag_sends.append(r)
            for d in range(1, N_DEV):
                src = (my + d) % N_DEV
                pltpu.make_async_remote_copy(
                    src_ref=xbuf.at[pl.ds(src * ROWS, ROWS), :],
                    dst_ref=xbuf.at[pl.ds(src * ROWS, ROWS), :],
                    send_sem=ag_send.at[src],
                    recv_sem=ag_recv.at[src],
                    device_id=(src,),
                    device_id_type=pl.DeviceIdType.MESH,
                ).wait_recv()
            for r in ag_sends:
                r.wait_send()

        for c in fetch(0) + fetch(1):
            c.start()
        xbuf[...] = x_ref[...].astype(jnp.bfloat16)

        for t in range(T):
            k, j = divmod(t, J)
            if t + 2 < T:
                for c in fetch(t + 2):
                    c.start()
            for c in fetch(t):
                c.wait()
            s = t % NBUF
            h = jnp.dot(xbuf[...], wa[s].astype(jnp.bfloat16),
                        preferred_element_type=jnp.float32)
            h = jnp.maximum(h, 0.0).astype(jnp.bfloat16)
            p = jnp.dot(h, wb[s].astype(jnp.bfloat16),
                        preferred_element_type=jnp.float32)
            if j == 0:
                acc[...] = p
            else:
                acc[...] += p
            if j == J - 1:
                comm(k)

    return pl.pallas_call(
        body,
        in_specs=[
            pl.BlockSpec(memory_space=pltpu.VMEM),
            pl.BlockSpec(memory_space=pl.ANY),
            pl.BlockSpec(memory_space=pl.ANY),
            pl.BlockSpec(memory_space=pl.ANY),
            pl.BlockSpec(memory_space=pl.ANY),
            pl.BlockSpec(memory_space=pl.ANY),
            pl.BlockSpec(memory_space=pl.ANY),
        ],
        out_specs=pl.BlockSpec(memory_space=pltpu.VMEM),
        out_shape=jax.ShapeDtypeStruct((ROWS, D), jnp.float32),
        scratch_shapes=[
            pltpu.VMEM((NBUF, D, BH), jnp.float32),
            pltpu.VMEM((NBUF, BH, D), jnp.float32),
            pltpu.SemaphoreType.DMA((NBUF,)),
            pltpu.SemaphoreType.DMA((NBUF,)),
            pltpu.VMEM((B, D), jnp.bfloat16),
            pltpu.VMEM((B, D), jnp.float32),
            pltpu.VMEM((B, D), jnp.bfloat16),
            pltpu.VMEM((N_DEV, ROWS, D), jnp.bfloat16),
            pltpu.SemaphoreType.DMA((N_DEV,)),
            pltpu.SemaphoreType.DMA((N_DEV,)),
            pltpu.SemaphoreType.DMA((N_DEV,)),
            pltpu.SemaphoreType.DMA((N_DEV,)),
        ],
    )(x, Win0, Wout0, Win1, Wout1, Win2, Wout2)
